# baseline (speedup 1.0000x reference)
import functools

import jax
import jax.numpy as jnp
from jax import lax
from jax.experimental import pallas as pl
from jax.experimental.pallas import tpu as pltpu
from jax.experimental.pallas import tpu_sc as plsc

B = 4096
V = 100000
K = 64

_info = plsc.get_sparse_core_info()
_NC, _NS = _info.num_cores, _info.num_subcores
NW = _NC * _NS
R = B // NW
NEG_W = R * K


def _sc_gather(outflat, target, noise_flat):
    mesh = plsc.VectorSubcoreMesh(core_axis_name="c", subcore_axis_name="s")

    @functools.partial(
        pl.kernel,
        mesh=mesh,
        out_type=[
            jax.ShapeDtypeStruct((B,), jnp.float32),
            jax.ShapeDtypeStruct((B * K,), jnp.float32),
        ],
        scratch_types=[
            pltpu.VMEM((R,), jnp.int32),
            pltpu.VMEM((NEG_W,), jnp.int32),
            pltpu.VMEM((R,), jnp.int32),
            pltpu.VMEM((NEG_W,), jnp.int32),
            pltpu.VMEM((R,), jnp.float32),
            pltpu.VMEM((NEG_W,), jnp.float32),
            pltpu.SemaphoreType.DMA,
            pltpu.SemaphoreType.DMA,
        ],
    )
    def k(out_hbm, tgt_hbm, noise_hbm, pos_hbm, neg_hbm,
          tgt_v, noise_v, ipos_v, ineg_v, pos_v, neg_v, sem0, sem1):
        wid = lax.axis_index("s") * _NC + lax.axis_index("c")
        base = wid * R
        pltpu.sync_copy(tgt_hbm.at[pl.ds(base, R)], tgt_v)
        pltpu.sync_copy(noise_hbm.at[pl.ds(base * K, NEG_W)], noise_v)

        iota = lax.iota(jnp.int32, 16)
        for c in range(R // 16):
            rows = (base + c * 16) + iota
            ipos_v[pl.ds(c * 16, 16)] = tgt_v[pl.ds(c * 16, 16)] + rows * V

        def body(i, carry):
            row = base + i // (K // 16)
            ineg_v[pl.ds(i * 16, 16)] = noise_v[pl.ds(i * 16, 16)] + row * V
            return carry
        lax.fori_loop(0, NEG_W // 16, body, 0)

        cp_pos = pltpu.async_copy(out_hbm.at[ipos_v], pos_v, sem0)
        cp_neg = pltpu.async_copy(out_hbm.at[ineg_v], neg_v, sem1)
        cp_pos.wait()
        cp_neg.wait()

        pltpu.sync_copy(pos_v, pos_hbm.at[pl.ds(base, R)])
        pltpu.sync_copy(neg_v, neg_hbm.at[pl.ds(base * K, NEG_W)])

    return k(outflat, target, noise_flat)


def _tc_loss_body(pos_ref, neg_ref, out_ref):
    p = pos_ref[...]
    n = neg_ref[...]
    lp = jnp.minimum(p, 0.0) - jnp.log1p(jnp.exp(-jnp.abs(p)))
    ln = jnp.minimum(-n, 0.0) - jnp.log1p(jnp.exp(-jnp.abs(n)))
    out_ref[0, 0] = -(jnp.sum(lp) + jnp.sum(ln)) / B


def _tc_loss(pos2d, neg2d):
    return pl.pallas_call(
        _tc_loss_body,
        out_shape=jax.ShapeDtypeStruct((1, 1), jnp.float32),
        out_specs=pl.BlockSpec(memory_space=pltpu.SMEM),
    )(pos2d, neg2d)


def kernel(output, target, noise):
    outflat = output.reshape(-1)
    pos, neg = _sc_gather(
        outflat,
        target.astype(jnp.int32),
        noise.reshape(-1).astype(jnp.int32),
    )
    loss = _tc_loss(pos.reshape(B // 128, 128), neg.reshape(B * K // 128, 128))
    return loss[0, 0]


# speedup vs baseline: 2.2544x; 2.2544x over previous
import functools

import jax
import jax.numpy as jnp
from jax import lax
from jax.experimental import pallas as pl
from jax.experimental.pallas import tpu as pltpu
from jax.experimental.pallas import tpu_sc as plsc

B = 4096
V = 100000
K = 64

_info = plsc.get_sparse_core_info()
_NC, _NS = _info.num_cores, _info.num_subcores
NW = _NC * _NS
R = B // NW
NEG_W = R * K
WAVE = 256
NWAVES = NEG_W // WAVE


def _sc_gather(out2d, target, noise_flat):
    mesh = plsc.VectorSubcoreMesh(core_axis_name="c", subcore_axis_name="s")

    @functools.partial(
        pl.kernel,
        mesh=mesh,
        out_type=[
            jax.ShapeDtypeStruct((B,), jnp.float32),
            jax.ShapeDtypeStruct((B * K,), jnp.float32),
        ],
        scratch_types=[
            pltpu.VMEM((R,), jnp.int32),
            pltpu.VMEM((NEG_W,), jnp.int32),
            pltpu.VMEM((R * 8,), jnp.int32),
            pltpu.VMEM((WAVE, 128), jnp.float32),
            pltpu.VMEM((WAVE, 128), jnp.float32),
            pltpu.VMEM((R,), jnp.float32),
            pltpu.VMEM((NEG_W,), jnp.float32),
            pltpu.SemaphoreType.DMA,
            pltpu.SemaphoreType.DMA,
        ],
        compiler_params=pltpu.CompilerParams(needs_layout_passes=False),
    )
    def k(x_hbm, tgt_hbm, noise_hbm, pos_hbm, neg_hbm,
          tgt_v, noise_v, rows_v, bufa, bufb, pscore_v, nscore_v,
          sema, semb):
        wid = lax.axis_index("s") * _NC + lax.axis_index("c")
        base = wid * R
        pltpu.sync_copy(tgt_hbm.at[pl.ds(base, R)], tgt_v)
        pltpu.sync_copy(noise_hbm.at[pl.ds(base * K, NEG_W)], noise_v)

        iota = lax.iota(jnp.int32, 16)
        for c in range(R // 2):
            rows_v[pl.ds(c * 16, 16)] = (base + c * 2) + (iota >> 3)

        bufs = (bufa, bufb)
        sems = (sema, semb)

        def fire(w, buf, sem):
            def body(i, carry):
                s = i * 16
                e0 = w * WAVE + s
                cols = noise_v[pl.ds(e0, 16)]
                r_local = e0 >> 6
                row_ref = rows_v.at[pl.ds(r_local * 8, 1)]
                for t in range(16):
                    start = pl.multiple_of((cols[t] >> 7) << 7, 128)
                    pltpu.async_copy(
                        x_hbm.at[row_ref, pl.ds(start, 128)],
                        buf.at[pl.ds(s + t, 1)],
                        sem,
                    )
                return carry
            lax.fori_loop(0, WAVE // 16, body, 0)

        def drain(buf, sem):
            pltpu.make_async_copy(
                x_hbm.at[pl.ds(0, WAVE), pl.ds(0, 128)], buf, sem
            ).wait()

        def extract(w, buf):
            def body(i, carry):
                s = i * 16
                lanes = noise_v[pl.ds(w * WAVE + s, 16)] & 127
                vals = plsc.load_gather(buf, [s + iota, lanes])
                nscore_v[pl.ds(w * WAVE + s, 16)] = vals
                return carry
            lax.fori_loop(0, WAVE // 16, body, 0)

        fire(0, bufs[0], sems[0])
        for w in range(NWAVES):
            if w + 1 < NWAVES:
                fire(w + 1, bufs[(w + 1) % 2], sems[(w + 1) % 2])
            drain(bufs[w % 2], sems[w % 2])
            extract(w, bufs[w % 2])

        def pfire(i, carry):
            s = i * 16
            cols = tgt_v[pl.ds(s, 16)]
            for t in range(16):
                start = pl.multiple_of((cols[t] >> 7) << 7, 128)
                pltpu.async_copy(
                    x_hbm.at[rows_v.at[pl.ds((s + t) * 8, 1)],
                             pl.ds(start, 128)],
                    bufa.at[pl.ds(s + t, 1)],
                    sema,
                )
            return carry
        lax.fori_loop(0, R // 16, pfire, 0)
        pltpu.make_async_copy(
            x_hbm.at[pl.ds(0, R), pl.ds(0, 128)],
            bufa.at[pl.ds(0, R)], sema,
        ).wait()

        def pextract(i, carry):
            s = i * 16
            lanes = tgt_v[pl.ds(s, 16)] & 127
            pscore_v[pl.ds(s, 16)] = plsc.load_gather(bufa, [s + iota, lanes])
            return carry
        lax.fori_loop(0, R // 16, pextract, 0)

        pltpu.sync_copy(pscore_v, pos_hbm.at[pl.ds(base, R)])
        pltpu.sync_copy(nscore_v, neg_hbm.at[pl.ds(base * K, NEG_W)])

    return k(out2d, target, noise_flat)


def _tc_loss_body(pos_ref, neg_ref, out_ref):
    p = pos_ref[...]
    n = neg_ref[...]
    lp = jnp.minimum(p, 0.0) - jnp.log1p(jnp.exp(-jnp.abs(p)))
    ln = jnp.minimum(-n, 0.0) - jnp.log1p(jnp.exp(-jnp.abs(n)))
    out_ref[0, 0] = -(jnp.sum(lp) + jnp.sum(ln)) / B


def _tc_loss(pos2d, neg2d):
    return pl.pallas_call(
        _tc_loss_body,
        out_shape=jax.ShapeDtypeStruct((1, 1), jnp.float32),
        out_specs=pl.BlockSpec(memory_space=pltpu.SMEM),
    )(pos2d, neg2d)


def kernel(output, target, noise):
    pos, neg = _sc_gather(
        output,
        target.astype(jnp.int32),
        noise.reshape(-1).astype(jnp.int32),
    )
    loss = _tc_loss(pos.reshape(B // 128, 128), neg.reshape(B * K // 128, 128))
    return loss[0, 0]


# speedup vs baseline: 2.2677x; 1.0059x over previous
import functools

import jax
import jax.numpy as jnp
from jax import lax
from jax.experimental import pallas as pl
from jax.experimental.pallas import tpu as pltpu
from jax.experimental.pallas import tpu_sc as plsc

B = 4096
V = 100000
K = 64

_info = plsc.get_sparse_core_info()
_NC, _NS = _info.num_cores, _info.num_subcores
NW = _NC * _NS
R = B // NW
NEG_W = R * K
WAVE = 256
NWAVES = NEG_W // WAVE


def _sc_gather(out2d, target, noise_flat):
    mesh = plsc.VectorSubcoreMesh(core_axis_name="c", subcore_axis_name="s")

    @functools.partial(
        pl.kernel,
        mesh=mesh,
        out_type=[
            jax.ShapeDtypeStruct((B,), jnp.float32),
            jax.ShapeDtypeStruct((B * K,), jnp.float32),
        ],
        scratch_types=[
            pltpu.VMEM((R,), jnp.int32),
            pltpu.VMEM((NEG_W,), jnp.int32),
            pltpu.VMEM((R * 8,), jnp.int32),
            pltpu.VMEM((WAVE, 128), jnp.float32),
            pltpu.VMEM((WAVE, 128), jnp.float32),
            pltpu.VMEM((R,), jnp.float32),
            pltpu.VMEM((NEG_W,), jnp.float32),
            pltpu.SemaphoreType.DMA,
            pltpu.SemaphoreType.DMA,
        ],
        compiler_params=pltpu.CompilerParams(needs_layout_passes=False),
    )
    def k(x_hbm, tgt_hbm, noise_hbm, pos_hbm, neg_hbm,
          tgt_v, noise_v, rows_v, bufa, bufb, pscore_v, nscore_v,
          sema, semb):
        wid = lax.axis_index("s") * _NC + lax.axis_index("c")
        base = wid * R
        pltpu.sync_copy(tgt_hbm.at[pl.ds(base, R)], tgt_v)
        pltpu.sync_copy(noise_hbm.at[pl.ds(base * K, NEG_W)], noise_v)

        iota = lax.iota(jnp.int32, 16)
        for c in range(R // 2):
            rows_v[pl.ds(c * 16, 16)] = (base + c * 2) + (iota >> 3)


        def fire(w, buf, sem):
            def body(i, carry):
                s = i * 16
                e0 = w * WAVE + s
                cols = noise_v[pl.ds(e0, 16)]
                r_local = e0 >> 6
                row_ref = rows_v.at[pl.ds(r_local * 8, 1)]
                for t in range(16):
                    start = pl.multiple_of((cols[t] >> 7) << 7, 128)
                    pltpu.async_copy(
                        x_hbm.at[row_ref, pl.ds(start, 128)],
                        buf.at[pl.ds(s + t, 1)],
                        sem,
                    )
                return carry
            lax.fori_loop(0, WAVE // 16, body, 0)

        def drain(buf, sem):
            pltpu.make_async_copy(
                x_hbm.at[pl.ds(0, WAVE), pl.ds(0, 128)], buf, sem
            ).wait()

        def extract(w, buf):
            def body(i, carry):
                s = i * 16
                lanes = noise_v[pl.ds(w * WAVE + s, 16)] & 127
                vals = plsc.load_gather(buf, [s + iota, lanes])
                nscore_v[pl.ds(w * WAVE + s, 16)] = vals
                return carry
            lax.fori_loop(0, WAVE // 16, body, 0)

        fire(0, bufa, sema)

        def pair(k, carry):
            w = k * 2
            fire(w + 1, bufb, semb)
            drain(bufa, sema)
            extract(w, bufa)

            @pl.when(w + 2 < NWAVES)
            def _():
                fire(w + 2, bufa, sema)
            drain(bufb, semb)
            extract(w + 1, bufb)
            return carry
        lax.fori_loop(0, NWAVES // 2, pair, 0)

        def pfire(i, carry):
            s = i * 16
            cols = tgt_v[pl.ds(s, 16)]
            for t in range(16):
                start = pl.multiple_of((cols[t] >> 7) << 7, 128)
                pltpu.async_copy(
                    x_hbm.at[rows_v.at[pl.ds((s + t) * 8, 1)],
                             pl.ds(start, 128)],
                    bufa.at[pl.ds(s + t, 1)],
                    sema,
                )
            return carry
        lax.fori_loop(0, R // 16, pfire, 0)
        pltpu.make_async_copy(
            x_hbm.at[pl.ds(0, R), pl.ds(0, 128)],
            bufa.at[pl.ds(0, R)], sema,
        ).wait()

        def pextract(i, carry):
            s = i * 16
            lanes = tgt_v[pl.ds(s, 16)] & 127
            pscore_v[pl.ds(s, 16)] = plsc.load_gather(bufa, [s + iota, lanes])
            return carry
        lax.fori_loop(0, R // 16, pextract, 0)

        pltpu.sync_copy(pscore_v, pos_hbm.at[pl.ds(base, R)])
        pltpu.sync_copy(nscore_v, neg_hbm.at[pl.ds(base * K, NEG_W)])

    return k(out2d, target, noise_flat)


def _tc_loss_body(pos_ref, neg_ref, out_ref):
    p = pos_ref[...]
    n = neg_ref[...]
    lp = jnp.minimum(p, 0.0) - jnp.log1p(jnp.exp(-jnp.abs(p)))
    ln = jnp.minimum(-n, 0.0) - jnp.log1p(jnp.exp(-jnp.abs(n)))
    out_ref[0, 0] = -(jnp.sum(lp) + jnp.sum(ln)) / B


def _tc_loss(pos2d, neg2d):
    return pl.pallas_call(
        _tc_loss_body,
        out_shape=jax.ShapeDtypeStruct((1, 1), jnp.float32),
        out_specs=pl.BlockSpec(memory_space=pltpu.SMEM),
    )(pos2d, neg2d)


def kernel(output, target, noise):
    pos, neg = _sc_gather(
        output,
        target.astype(jnp.int32),
        noise.reshape(-1).astype(jnp.int32),
    )
    loss = _tc_loss(pos.reshape(B // 128, 128), neg.reshape(B * K // 128, 128))
    return loss[0, 0]
